# baseline (speedup 1.0000x reference)
import jax
import jax.numpy as jnp
from jax import lax
from jax.experimental import pallas as pl
from jax.experimental.pallas import tpu as pltpu
from jax.experimental.pallas import tpu_sc as plsc

N = 5000
P = 5120
NSUB = 16
CHUNK = P // NSUB
NVREG = CHUNK // 16
MAX_DET = 100
IOU_THRESH = 0.5
SCORE_THRESH = 0.05
NEG = float("-inf")
BIGI = 2 ** 30
KPUB = 4
KSEL = 8
ROW = 32
OUTROWS = MAX_DET + KSEL


def _butterfly(v, rots, op):
    for r in rots:
        v = op(v, v.at[r].get(mode="promise_in_bounds"))
    return v


def _insert_top4(vals, lins, v, linv):
    m1, m2, m3, m4 = vals
    l1, l2, l3, l4 = lins
    c1 = v > m1
    c2 = v > m2
    c3 = v > m3
    c4 = v > m4
    n4 = jnp.where(c3, m3, jnp.where(c4, v, m4))
    x4 = jnp.where(c3, l3, jnp.where(c4, linv, l4))
    n3 = jnp.where(c2, m2, jnp.where(c3, v, m3))
    x3 = jnp.where(c2, l2, jnp.where(c3, linv, l3))
    n2 = jnp.where(c1, m1, jnp.where(c2, v, m2))
    x2 = jnp.where(c1, l1, jnp.where(c2, linv, l2))
    n1 = jnp.where(c1, v, m1)
    x1 = jnp.where(c1, linv, l1)
    return (n1, n2, n3, n4), (x1, x2, x3, x4)


def _chain_extract(vals, lins, ksel, rots, neg_v, bigi_v, with_validity):
    depth = len(vals)
    cnt = jnp.zeros((16,), jnp.int32)
    vprev = neg_v == neg_v
    outs = []
    for _ in range(ksel):
        cur = neg_v
        curl = bigi_v
        for kk in range(depth - 1, -1, -1):
            sel = cnt == kk
            cur = jnp.where(sel, vals[kk], cur)
            curl = jnp.where(sel, lins[kk], curl)
        g = _butterfly(cur, rots, jnp.maximum)
        li = _butterfly(jnp.where(cur == g, curl, bigi_v), rots, jnp.minimum)
        if with_validity:
            hid = jnp.where(cnt >= depth, vals[depth - 1], neg_v)
            hmax = _butterfly(hid, rots, jnp.maximum)
            vprev = vprev & (g > hmax)
            outs.append((g, li, vprev))
        else:
            outs.append((g, li))
        cnt = cnt + jnp.where(curl == li, 1, 0)
    return outs


def _nms_kernel(xs_hbm, sc_hbm, out_hbm,
                x1_v, y1_v, x2_v, y2_v, areas_v, work_v,
                stage, loc, outbuf, sh, sem):
    c = lax.axis_index("c")
    s = lax.axis_index("s")
    base = pl.multiple_of(s * CHUNK, CHUNK)
    lane = lax.iota(jnp.int32, 16)
    rots = [jnp.bitwise_and(lane + sh, 15) for sh in (1, 2, 4, 8)]
    neg_v = jnp.full((16,), NEG, jnp.float32)
    zero_v = jnp.zeros((16,), jnp.float32)
    bigi_v = jnp.full((16,), BIGI, jnp.int32)
    laneb = lane + base

    cps = [pltpu.async_copy(xs_hbm.at[0], x1_v, sem),
           pltpu.async_copy(xs_hbm.at[1], y1_v, sem),
           pltpu.async_copy(xs_hbm.at[2], x2_v, sem),
           pltpu.async_copy(xs_hbm.at[3], y2_v, sem),
           pltpu.async_copy(sc_hbm.at[pl.ds(base, CHUNK)], work_v, sem)]
    for cp in cps:
        cp.wait()

    def area_body(jj, carry):
        for u in range(8):
            o = (jj * 8 + u) * 16
            w = jnp.maximum(x2_v[pl.ds(o, 16)] - x1_v[pl.ds(o, 16)], 0.0)
            h = jnp.maximum(y2_v[pl.ds(o, 16)] - y1_v[pl.ds(o, 16)], 0.0)
            areas_v[pl.ds(o, 16)] = w * h
        return carry
    lax.fori_loop(0, P // 128, area_body, 0)

    def zero_body(r, carry):
        outbuf[pl.ds(r * 16, 16)] = zero_v
        return carry
    lax.fori_loop(0, OUTROWS, zero_body, 0)

    vals = (neg_v, neg_v, neg_v, neg_v)
    lins = (bigi_v, bigi_v, bigi_v, bigi_v)
    for j in range(NVREG):
        o = j * 16
        v = work_v[pl.ds(o, 16)]
        v = jnp.where(v > SCORE_THRESH, v, neg_v)
        work_v[pl.ds(o, 16)] = v
        vals, lins = _insert_top4(vals, lins, v, laneb + o)

    def cond_fun(carry):
        ptr, vals, lins, parity = carry
        g = _butterfly(vals[0], rots, jnp.maximum)
        return (ptr < MAX_DET) & (g[0] > NEG)

    def body_fun(carry):
        ptr, vals, lins, parity = carry

        tile_cands = _chain_extract(vals, lins, KPUB, rots, neg_v, bigi_v,
                                    with_validity=False)
        q = jnp.right_shift(lane, 2)
        for half in range(2):
            (ga, lia), (gb, lib) = tile_cands[2 * half], tile_cands[2 * half + 1]
            packed = jnp.where(q == 0, ga,
                     jnp.where(q == 1, plsc.bitcast(lia, jnp.float32),
                     jnp.where(q == 2, gb, plsc.bitcast(lib, jnp.float32))))
            stage[pl.ds(half * 16, 16)] = packed
        pltpu.sync_copy(stage.at[pl.ds(0, ROW)],
                        sh.at[pl.ds(parity * (NSUB * ROW) + s * ROW, ROW)])
        plsc.subcore_barrier()
        pltpu.sync_copy(sh.at[pl.ds(parity * (NSUB * ROW), NSUB * ROW)], loc)

        l32 = lane * ROW
        gvals = []
        glins = []
        for t in range(KPUB):
            gvals.append(plsc.load_gather(loc, [l32 + (8 * t)]))
            glins.append(plsc.bitcast(
                plsc.load_gather(loc, [l32 + (8 * t + 4)]), jnp.int32))
        cands = _chain_extract(gvals, glins, KSEL, rots, neg_v, bigi_v,
                               with_validity=True)

        gidx = [jnp.where(g > neg_v, li, 0) for g, li, _v in cands]
        bx1 = [plsc.load_gather(x1_v, [li]) for li in gidx]
        by1 = [plsc.load_gather(y1_v, [li]) for li in gidx]
        bx2 = [plsc.load_gather(x2_v, [li]) for li in gidx]
        by2 = [plsc.load_gather(y2_v, [li]) for li in gidx]
        bar = [plsc.load_gather(areas_v, [li]) for li in gidx]

        pend = {}
        for a in range(KSEL):
            for b in range(a + 1, KSEL):
                ltx = jnp.maximum(bx1[a], bx1[b])
                lty = jnp.maximum(by1[a], by1[b])
                rbx = jnp.minimum(bx2[a], bx2[b])
                rby = jnp.minimum(by2[a], by2[b])
                w = jnp.maximum(rbx - ltx, 0.0)
                h = jnp.maximum(rby - lty, 0.0)
                inter = w * h
                den = jnp.maximum((bar[a] + bar[b]) - inter, 1e-9)
                pend[(a, b)] = (inter, den)
        iou = {ab: inter / den for ab, (inter, den) in pend.items()}
        acc = [cands[0][2] & (cands[0][0] > neg_v)]
        for b in range(1, KSEL):
            ok = cands[b][2] & (cands[b][0] > neg_v)
            for a in range(b):
                ok = ok & ~(acc[a] & (iou[(a, b)] > IOU_THRESH))
            acc.append(ok)

        nvals = (neg_v, neg_v, neg_v, neg_v)
        nlins = (bigi_v, bigi_v, bigi_v, bigi_v)
        for j in range(NVREG):
            o = j * 16
            ao = base + o
            x1o = x1_v[pl.ds(ao, 16)]
            y1o = y1_v[pl.ds(ao, 16)]
            x2o = x2_v[pl.ds(ao, 16)]
            y2o = y2_v[pl.ds(ao, 16)]
            aro = areas_v[pl.ds(ao, 16)]
            pend2 = []
            for k in range(KSEL):
                ltx = jnp.maximum(bx1[k], x1o)
                lty = jnp.maximum(by1[k], y1o)
                rbx = jnp.minimum(bx2[k], x2o)
                rby = jnp.minimum(by2[k], y2o)
                w = jnp.maximum(rbx - ltx, 0.0)
                h = jnp.maximum(rby - lty, 0.0)
                inter = w * h
                den = jnp.maximum((bar[k] + aro) - inter, 1e-9)
                pend2.append(inter / den)
            linv = laneb + o
            sup = acc[0] & ((pend2[0] > IOU_THRESH) | (linv == cands[0][1]))
            for k in range(1, KSEL):
                sup = sup | (acc[k] & ((pend2[k] > IOU_THRESH)
                                       | (linv == cands[k][1])))
            wv = jnp.where(sup, neg_v, work_v[pl.ds(o, 16)])
            work_v[pl.ds(o, 16)] = wv
            nvals, nlins = _insert_top4(nvals, nlins, wv, linv)

        a_s = [jnp.where(acc[k], 1, 0)[0] for k in range(1, KSEL)]
        pos = [ptr]
        run = ptr + 1
        for k in range(1, KSEL):
            pos.append(jnp.where(a_s[k - 1] == 1, run, MAX_DET))
            run = run + a_s[k - 1]
        for k in range(KSEL):
            g, _li, _v = cands[k]
            row = jnp.where(lane == 0, bx1[k],
                  jnp.where(lane == 1, by1[k],
                  jnp.where(lane == 2, bx2[k],
                  jnp.where(lane == 3, by2[k],
                  jnp.where(lane == 4, g, zero_v)))))
            outbuf[pl.ds(pos[k] * 16, 16)] = row
        return (run, nvals, nlins, 1 - parity)

    lax.while_loop(cond_fun, body_fun,
                   (jnp.int32(0), vals, lins, jnp.int32(0)))

    @pl.when((c == 0) & (s == 0))
    def _():
        pltpu.sync_copy(outbuf.at[pl.ds(0, MAX_DET * 16)], out_hbm)


@jax.jit
def _nms_sc(xs, sc):
    mesh = plsc.VectorSubcoreMesh(core_axis_name="c", subcore_axis_name="s")
    f = pl.kernel(
        _nms_kernel,
        out_type=jax.ShapeDtypeStruct((MAX_DET * 16,), jnp.float32),
        mesh=mesh,
        compiler_params=pltpu.CompilerParams(needs_layout_passes=False),
        scratch_types=[
            pltpu.VMEM((P,), jnp.float32),
            pltpu.VMEM((P,), jnp.float32),
            pltpu.VMEM((P,), jnp.float32),
            pltpu.VMEM((P,), jnp.float32),
            pltpu.VMEM((P,), jnp.float32),
            pltpu.VMEM((CHUNK,), jnp.float32),
            pltpu.VMEM((ROW,), jnp.float32),
            pltpu.VMEM((NSUB * ROW,), jnp.float32),
            pltpu.VMEM((OUTROWS * 16,), jnp.float32),
            pltpu.VMEM_SHARED((2 * NSUB * ROW,), jnp.float32),
            pltpu.SemaphoreType.DMA,
        ],
    )
    return f(xs, sc)


def kernel(boxes, scores):
    xs = jnp.zeros((4, P), jnp.float32).at[:, :N].set(boxes.T)
    sc = jnp.full((P,), -1.0, jnp.float32).at[:N].set(scores)
    out = _nms_sc(xs, sc)
    return out.reshape(MAX_DET, 16)[:, :5]


# speedup vs baseline: 1.0100x; 1.0100x over previous
import jax
import jax.numpy as jnp
from jax import lax
from jax.experimental import pallas as pl
from jax.experimental.pallas import tpu as pltpu
from jax.experimental.pallas import tpu_sc as plsc

N = 5000
P = 5120
NSUB = 16
CHUNK = P // NSUB
NVREG = CHUNK // 16
MAX_DET = 100
IOU_THRESH = 0.5
SCORE_THRESH = 0.05
NEG = float("-inf")
BIGI = 2 ** 30
KPUB = 4
KSEL = 6
ROW = 32
OUTROWS = MAX_DET + KSEL


def _butterfly(v, rots, op):
    for r in rots:
        v = op(v, v.at[r].get(mode="promise_in_bounds"))
    return v


def _insert_top4(vals, lins, v, linv):
    m1, m2, m3, m4 = vals
    l1, l2, l3, l4 = lins
    c1 = v > m1
    c2 = v > m2
    c3 = v > m3
    c4 = v > m4
    n4 = jnp.where(c3, m3, jnp.where(c4, v, m4))
    x4 = jnp.where(c3, l3, jnp.where(c4, linv, l4))
    n3 = jnp.where(c2, m2, jnp.where(c3, v, m3))
    x3 = jnp.where(c2, l2, jnp.where(c3, linv, l3))
    n2 = jnp.where(c1, m1, jnp.where(c2, v, m2))
    x2 = jnp.where(c1, l1, jnp.where(c2, linv, l2))
    n1 = jnp.where(c1, v, m1)
    x1 = jnp.where(c1, linv, l1)
    return (n1, n2, n3, n4), (x1, x2, x3, x4)


def _chain_extract(vals, lins, ksel, rots, neg_v, bigi_v, with_validity):
    depth = len(vals)
    cnt = jnp.zeros((16,), jnp.int32)
    vprev = neg_v == neg_v
    outs = []
    for _ in range(ksel):
        cur = neg_v
        curl = bigi_v
        for kk in range(depth - 1, -1, -1):
            sel = cnt == kk
            cur = jnp.where(sel, vals[kk], cur)
            curl = jnp.where(sel, lins[kk], curl)
        g = _butterfly(cur, rots, jnp.maximum)
        li = _butterfly(jnp.where(cur == g, curl, bigi_v), rots, jnp.minimum)
        if with_validity:
            hid = jnp.where(cnt >= depth, vals[depth - 1], neg_v)
            hmax = _butterfly(hid, rots, jnp.maximum)
            vprev = vprev & (g > hmax)
            outs.append((g, li, vprev))
        else:
            outs.append((g, li))
        cnt = cnt + jnp.where(curl == li, 1, 0)
    return outs


def _nms_kernel(xs_hbm, sc_hbm, out_hbm,
                x1_v, y1_v, x2_v, y2_v, areas_v, work_v,
                stage, loc, outbuf, sh, sem):
    c = lax.axis_index("c")
    s = lax.axis_index("s")
    base = pl.multiple_of(s * CHUNK, CHUNK)
    lane = lax.iota(jnp.int32, 16)
    rots = [jnp.bitwise_and(lane + sh, 15) for sh in (1, 2, 4, 8)]
    neg_v = jnp.full((16,), NEG, jnp.float32)
    zero_v = jnp.zeros((16,), jnp.float32)
    bigi_v = jnp.full((16,), BIGI, jnp.int32)
    laneb = lane + base

    cps = [pltpu.async_copy(xs_hbm.at[0], x1_v, sem),
           pltpu.async_copy(xs_hbm.at[1], y1_v, sem),
           pltpu.async_copy(xs_hbm.at[2], x2_v, sem),
           pltpu.async_copy(xs_hbm.at[3], y2_v, sem),
           pltpu.async_copy(sc_hbm.at[pl.ds(base, CHUNK)], work_v, sem)]
    for cp in cps:
        cp.wait()

    def area_body(jj, carry):
        for u in range(8):
            o = (jj * 8 + u) * 16
            w = jnp.maximum(x2_v[pl.ds(o, 16)] - x1_v[pl.ds(o, 16)], 0.0)
            h = jnp.maximum(y2_v[pl.ds(o, 16)] - y1_v[pl.ds(o, 16)], 0.0)
            areas_v[pl.ds(o, 16)] = w * h
        return carry
    lax.fori_loop(0, P // 128, area_body, 0)

    def zero_body(r, carry):
        outbuf[pl.ds(r * 16, 16)] = zero_v
        return carry
    lax.fori_loop(0, OUTROWS, zero_body, 0)

    vals = (neg_v, neg_v, neg_v, neg_v)
    lins = (bigi_v, bigi_v, bigi_v, bigi_v)
    for j in range(NVREG):
        o = j * 16
        v = work_v[pl.ds(o, 16)]
        v = jnp.where(v > SCORE_THRESH, v, neg_v)
        work_v[pl.ds(o, 16)] = v
        vals, lins = _insert_top4(vals, lins, v, laneb + o)

    def cond_fun(carry):
        ptr, vals, lins, parity = carry
        g = _butterfly(vals[0], rots, jnp.maximum)
        return (ptr < MAX_DET) & (g[0] > NEG)

    def body_fun(carry):
        ptr, vals, lins, parity = carry

        tile_cands = _chain_extract(vals, lins, KPUB, rots, neg_v, bigi_v,
                                    with_validity=False)
        q = jnp.right_shift(lane, 2)
        for half in range(2):
            (ga, lia), (gb, lib) = tile_cands[2 * half], tile_cands[2 * half + 1]
            packed = jnp.where(q == 0, ga,
                     jnp.where(q == 1, plsc.bitcast(lia, jnp.float32),
                     jnp.where(q == 2, gb, plsc.bitcast(lib, jnp.float32))))
            stage[pl.ds(half * 16, 16)] = packed
        pltpu.sync_copy(stage.at[pl.ds(0, ROW)],
                        sh.at[pl.ds(parity * (NSUB * ROW) + s * ROW, ROW)])
        plsc.subcore_barrier()
        pltpu.sync_copy(sh.at[pl.ds(parity * (NSUB * ROW), NSUB * ROW)], loc)

        l32 = lane * ROW
        gvals = []
        glins = []
        for t in range(KPUB):
            gvals.append(plsc.load_gather(loc, [l32 + (8 * t)]))
            glins.append(plsc.bitcast(
                plsc.load_gather(loc, [l32 + (8 * t + 4)]), jnp.int32))
        cands = _chain_extract(gvals, glins, KSEL, rots, neg_v, bigi_v,
                               with_validity=True)

        gidx = [jnp.where(g > neg_v, li, 0) for g, li, _v in cands]
        bx1 = [plsc.load_gather(x1_v, [li]) for li in gidx]
        by1 = [plsc.load_gather(y1_v, [li]) for li in gidx]
        bx2 = [plsc.load_gather(x2_v, [li]) for li in gidx]
        by2 = [plsc.load_gather(y2_v, [li]) for li in gidx]
        bar = [plsc.load_gather(areas_v, [li]) for li in gidx]

        pend = {}
        for a in range(KSEL):
            for b in range(a + 1, KSEL):
                ltx = jnp.maximum(bx1[a], bx1[b])
                lty = jnp.maximum(by1[a], by1[b])
                rbx = jnp.minimum(bx2[a], bx2[b])
                rby = jnp.minimum(by2[a], by2[b])
                w = jnp.maximum(rbx - ltx, 0.0)
                h = jnp.maximum(rby - lty, 0.0)
                inter = w * h
                den = jnp.maximum((bar[a] + bar[b]) - inter, 1e-9)
                pend[(a, b)] = (inter, den)
        iou = {ab: inter / den for ab, (inter, den) in pend.items()}
        acc = [cands[0][2] & (cands[0][0] > neg_v)]
        for b in range(1, KSEL):
            ok = cands[b][2] & (cands[b][0] > neg_v)
            for a in range(b):
                ok = ok & ~(acc[a] & (iou[(a, b)] > IOU_THRESH))
            acc.append(ok)

        nvals = (neg_v, neg_v, neg_v, neg_v)
        nlins = (bigi_v, bigi_v, bigi_v, bigi_v)
        for j in range(NVREG):
            o = j * 16
            ao = base + o
            x1o = x1_v[pl.ds(ao, 16)]
            y1o = y1_v[pl.ds(ao, 16)]
            x2o = x2_v[pl.ds(ao, 16)]
            y2o = y2_v[pl.ds(ao, 16)]
            aro = areas_v[pl.ds(ao, 16)]
            pend2 = []
            for k in range(KSEL):
                ltx = jnp.maximum(bx1[k], x1o)
                lty = jnp.maximum(by1[k], y1o)
                rbx = jnp.minimum(bx2[k], x2o)
                rby = jnp.minimum(by2[k], y2o)
                w = jnp.maximum(rbx - ltx, 0.0)
                h = jnp.maximum(rby - lty, 0.0)
                inter = w * h
                den = jnp.maximum((bar[k] + aro) - inter, 1e-9)
                pend2.append(inter / den)
            linv = laneb + o
            sup = acc[0] & ((pend2[0] > IOU_THRESH) | (linv == cands[0][1]))
            for k in range(1, KSEL):
                sup = sup | (acc[k] & ((pend2[k] > IOU_THRESH)
                                       | (linv == cands[k][1])))
            wv = jnp.where(sup, neg_v, work_v[pl.ds(o, 16)])
            work_v[pl.ds(o, 16)] = wv
            nvals, nlins = _insert_top4(nvals, nlins, wv, linv)

        a_s = [jnp.where(acc[k], 1, 0)[0] for k in range(1, KSEL)]
        pos = [ptr]
        run = ptr + 1
        for k in range(1, KSEL):
            pos.append(jnp.where(a_s[k - 1] == 1, run, MAX_DET))
            run = run + a_s[k - 1]
        for k in range(KSEL):
            g, _li, _v = cands[k]
            row = jnp.where(lane == 0, bx1[k],
                  jnp.where(lane == 1, by1[k],
                  jnp.where(lane == 2, bx2[k],
                  jnp.where(lane == 3, by2[k],
                  jnp.where(lane == 4, g, zero_v)))))
            outbuf[pl.ds(pos[k] * 16, 16)] = row
        return (run, nvals, nlins, 1 - parity)

    lax.while_loop(cond_fun, body_fun,
                   (jnp.int32(0), vals, lins, jnp.int32(0)))

    @pl.when((c == 0) & (s == 0))
    def _():
        pltpu.sync_copy(outbuf.at[pl.ds(0, MAX_DET * 16)], out_hbm)


@jax.jit
def _nms_sc(xs, sc):
    mesh = plsc.VectorSubcoreMesh(core_axis_name="c", subcore_axis_name="s")
    f = pl.kernel(
        _nms_kernel,
        out_type=jax.ShapeDtypeStruct((MAX_DET * 16,), jnp.float32),
        mesh=mesh,
        compiler_params=pltpu.CompilerParams(needs_layout_passes=False),
        scratch_types=[
            pltpu.VMEM((P,), jnp.float32),
            pltpu.VMEM((P,), jnp.float32),
            pltpu.VMEM((P,), jnp.float32),
            pltpu.VMEM((P,), jnp.float32),
            pltpu.VMEM((P,), jnp.float32),
            pltpu.VMEM((CHUNK,), jnp.float32),
            pltpu.VMEM((ROW,), jnp.float32),
            pltpu.VMEM((NSUB * ROW,), jnp.float32),
            pltpu.VMEM((OUTROWS * 16,), jnp.float32),
            pltpu.VMEM_SHARED((2 * NSUB * ROW,), jnp.float32),
            pltpu.SemaphoreType.DMA,
        ],
    )
    return f(xs, sc)


def kernel(boxes, scores):
    xs = jnp.zeros((4, P), jnp.float32).at[:, :N].set(boxes.T)
    sc = jnp.full((P,), -1.0, jnp.float32).at[:N].set(scores)
    out = _nms_sc(xs, sc)
    return out.reshape(MAX_DET, 16)[:, :5]


# speedup vs baseline: 1.0435x; 1.0332x over previous
import jax
import jax.numpy as jnp
from jax import lax
from jax.experimental import pallas as pl
from jax.experimental.pallas import tpu as pltpu
from jax.experimental.pallas import tpu_sc as plsc

N = 5000
P = 5120
NSUB = 16
CHUNK = P // NSUB
NVREG = CHUNK // 16
MAX_DET = 100
IOU_THRESH = 0.5
SCORE_THRESH = 0.05
NEG = float("-inf")
BIGI = 2 ** 30
K = 4
ROW = 32
OUTROWS = MAX_DET + K


def _butterfly(v, rots, op):
    for r in rots:
        v = op(v, v.at[r].get(mode="promise_in_bounds"))
    return v


def _insert_top4(vals, lins, v, linv):
    m1, m2, m3, m4 = vals
    l1, l2, l3, l4 = lins
    c1 = v > m1
    c2 = v > m2
    c3 = v > m3
    c4 = v > m4
    n4 = jnp.where(c3, m3, jnp.where(c4, v, m4))
    x4 = jnp.where(c3, l3, jnp.where(c4, linv, l4))
    n3 = jnp.where(c2, m2, jnp.where(c3, v, m3))
    x3 = jnp.where(c2, l2, jnp.where(c3, linv, l3))
    n2 = jnp.where(c1, m1, jnp.where(c2, v, m2))
    x2 = jnp.where(c1, l1, jnp.where(c2, linv, l2))
    n1 = jnp.where(c1, v, m1)
    x1 = jnp.where(c1, linv, l1)
    return (n1, n2, n3, n4), (x1, x2, x3, x4)


def _chain_top4(vals, lins, rots, neg_v, bigi_v):
    cnt = jnp.zeros((16,), jnp.int32)
    outs = []
    for _ in range(K):
        cur = neg_v
        curl = bigi_v
        for kk in range(len(vals) - 1, -1, -1):
            sel = cnt == kk
            cur = jnp.where(sel, vals[kk], cur)
            curl = jnp.where(sel, lins[kk], curl)
        g = _butterfly(cur, rots, jnp.maximum)
        li = _butterfly(jnp.where(cur == g, curl, bigi_v), rots, jnp.minimum)
        cnt = cnt + jnp.where(curl == li, 1, 0)
        outs.append((g, li))
    return outs


def _nms_kernel(xs_hbm, sc_hbm, out_hbm,
                x1_v, y1_v, x2_v, y2_v, areas_v, work_v,
                stage, loc, outbuf, sh, sem):
    c = lax.axis_index("c")
    s = lax.axis_index("s")
    base = pl.multiple_of(s * CHUNK, CHUNK)
    lane = lax.iota(jnp.int32, 16)
    rots = [jnp.bitwise_and(lane + sh, 15) for sh in (1, 2, 4, 8)]
    neg_v = jnp.full((16,), NEG, jnp.float32)
    zero_v = jnp.zeros((16,), jnp.float32)
    bigi_v = jnp.full((16,), BIGI, jnp.int32)
    laneb = lane + base

    cps = [pltpu.async_copy(xs_hbm.at[0], x1_v, sem),
           pltpu.async_copy(xs_hbm.at[1], y1_v, sem),
           pltpu.async_copy(xs_hbm.at[2], x2_v, sem),
           pltpu.async_copy(xs_hbm.at[3], y2_v, sem),
           pltpu.async_copy(sc_hbm.at[pl.ds(base, CHUNK)], work_v, sem)]
    for cp in cps:
        cp.wait()

    def area_body(jj, carry):
        for u in range(8):
            o = (jj * 8 + u) * 16
            w = jnp.maximum(x2_v[pl.ds(o, 16)] - x1_v[pl.ds(o, 16)], 0.0)
            h = jnp.maximum(y2_v[pl.ds(o, 16)] - y1_v[pl.ds(o, 16)], 0.0)
            areas_v[pl.ds(o, 16)] = w * h
        return carry
    lax.fori_loop(0, P // 128, area_body, 0)

    def zero_body(r, carry):
        outbuf[pl.ds(r * 16, 16)] = zero_v
        return carry
    lax.fori_loop(0, OUTROWS, zero_body, 0)

    vals = (neg_v, neg_v, neg_v, neg_v)
    lins = (bigi_v, bigi_v, bigi_v, bigi_v)
    for j in range(NVREG):
        o = j * 16
        v = work_v[pl.ds(o, 16)]
        v = jnp.where(v > SCORE_THRESH, v, neg_v)
        work_v[pl.ds(o, 16)] = v
        vals, lins = _insert_top4(vals, lins, v, laneb + o)

    def cond_fun(carry):
        ptr, vals, lins, parity, ppos, prows = carry
        g = _butterfly(vals[0], rots, jnp.maximum)
        return (ptr < MAX_DET) & (g[0] > NEG)

    def body_fun(carry):
        ptr, vals, lins, parity, ppos, prows = carry

        tile_cands = _chain_top4(vals, lins, rots, neg_v, bigi_v)
        q = jnp.right_shift(lane, 2)
        for half in range(2):
            (ga, lia), (gb, lib) = tile_cands[2 * half], tile_cands[2 * half + 1]
            packed = jnp.where(q == 0, ga,
                     jnp.where(q == 1, plsc.bitcast(lia, jnp.float32),
                     jnp.where(q == 2, gb, plsc.bitcast(lib, jnp.float32))))
            stage[pl.ds(half * 16, 16)] = packed
        pltpu.sync_copy(stage.at[pl.ds(0, ROW)],
                        sh.at[pl.ds(parity * (NSUB * ROW) + s * ROW, ROW)])
        for k in range(K):
            outbuf[pl.ds(ppos[k] * 16, 16)] = prows[k]
        plsc.subcore_barrier()
        pltpu.sync_copy(sh.at[pl.ds(parity * (NSUB * ROW), NSUB * ROW)], loc)

        l32 = lane * ROW
        gvals = []
        glins = []
        for t in range(K):
            gvals.append(plsc.load_gather(loc, [l32 + (8 * t)]))
            glins.append(plsc.bitcast(
                plsc.load_gather(loc, [l32 + (8 * t + 4)]), jnp.int32))
        cands = _chain_top4(gvals, glins, rots, neg_v, bigi_v)

        gidx = [jnp.where(g > neg_v, li, 0) for g, li in cands]
        bx1 = [plsc.load_gather(x1_v, [li]) for li in gidx]
        by1 = [plsc.load_gather(y1_v, [li]) for li in gidx]
        bx2 = [plsc.load_gather(x2_v, [li]) for li in gidx]
        by2 = [plsc.load_gather(y2_v, [li]) for li in gidx]
        bar = [plsc.load_gather(areas_v, [li]) for li in gidx]

        pend = {}
        for a in range(K):
            for b in range(a + 1, K):
                ltx = jnp.maximum(bx1[a], bx1[b])
                lty = jnp.maximum(by1[a], by1[b])
                rbx = jnp.minimum(bx2[a], bx2[b])
                rby = jnp.minimum(by2[a], by2[b])
                w = jnp.maximum(rbx - ltx, 0.0)
                h = jnp.maximum(rby - lty, 0.0)
                inter = w * h
                den = jnp.maximum((bar[a] + bar[b]) - inter, 1e-9)
                pend[(a, b)] = (inter, den)
        iou = {ab: inter / den for ab, (inter, den) in pend.items()}
        acc = [cands[0][0] > neg_v]
        for b in range(1, K):
            ok = cands[b][0] > neg_v
            for a in range(b):
                ok = ok & ~(acc[a] & (iou[(a, b)] > IOU_THRESH))
            acc.append(ok)

        nvals = (neg_v, neg_v, neg_v, neg_v)
        nlins = (bigi_v, bigi_v, bigi_v, bigi_v)
        for j in range(NVREG):
            o = j * 16
            ao = base + o
            x1o = x1_v[pl.ds(ao, 16)]
            y1o = y1_v[pl.ds(ao, 16)]
            x2o = x2_v[pl.ds(ao, 16)]
            y2o = y2_v[pl.ds(ao, 16)]
            aro = areas_v[pl.ds(ao, 16)]
            pend2 = []
            for k in range(K):
                ltx = jnp.maximum(bx1[k], x1o)
                lty = jnp.maximum(by1[k], y1o)
                rbx = jnp.minimum(bx2[k], x2o)
                rby = jnp.minimum(by2[k], y2o)
                w = jnp.maximum(rbx - ltx, 0.0)
                h = jnp.maximum(rby - lty, 0.0)
                inter = w * h
                den = jnp.maximum((bar[k] + aro) - inter, 1e-9)
                pend2.append(inter / den)
            linv = laneb + o
            sup = acc[0] & ((pend2[0] > IOU_THRESH) | (linv == cands[0][1]))
            for k in range(1, K):
                sup = sup | (acc[k] & ((pend2[k] > IOU_THRESH)
                                       | (linv == cands[k][1])))
            wv = jnp.where(sup, neg_v, work_v[pl.ds(o, 16)])
            work_v[pl.ds(o, 16)] = wv
            nvals, nlins = _insert_top4(nvals, nlins, wv, linv)

        a_s = [jnp.where(acc[k], 1, 0)[0] for k in range(1, K)]
        pos = [ptr]
        run = ptr + 1
        for k in range(1, K):
            pos.append(jnp.where(a_s[k - 1] == 1, run, jnp.int32(MAX_DET)))
            run = run + a_s[k - 1]
        rows = []
        for k in range(K):
            g, _li = cands[k]
            row = jnp.where(lane == 0, bx1[k],
                  jnp.where(lane == 1, by1[k],
                  jnp.where(lane == 2, bx2[k],
                  jnp.where(lane == 3, by2[k],
                  jnp.where(lane == 4, g, zero_v)))))
            rows.append(row)
        return (run, nvals, nlins, 1 - parity, tuple(pos), tuple(rows))

    fin = lax.while_loop(
        cond_fun, body_fun,
        (jnp.int32(0), vals, lins, jnp.int32(0),
         (jnp.int32(MAX_DET),) * K, (zero_v,) * K))
    fpos, frows = fin[4], fin[5]
    for k in range(K):
        outbuf[pl.ds(fpos[k] * 16, 16)] = frows[k]

    @pl.when((c == 0) & (s == 0))
    def _():
        pltpu.sync_copy(outbuf.at[pl.ds(0, MAX_DET * 16)], out_hbm)


@jax.jit
def _nms_sc(xs, sc):
    mesh = plsc.VectorSubcoreMesh(core_axis_name="c", subcore_axis_name="s")
    f = pl.kernel(
        _nms_kernel,
        out_type=jax.ShapeDtypeStruct((MAX_DET * 16,), jnp.float32),
        mesh=mesh,
        compiler_params=pltpu.CompilerParams(needs_layout_passes=False),
        scratch_types=[
            pltpu.VMEM((P,), jnp.float32),
            pltpu.VMEM((P,), jnp.float32),
            pltpu.VMEM((P,), jnp.float32),
            pltpu.VMEM((P,), jnp.float32),
            pltpu.VMEM((P,), jnp.float32),
            pltpu.VMEM((CHUNK,), jnp.float32),
            pltpu.VMEM((ROW,), jnp.float32),
            pltpu.VMEM((NSUB * ROW,), jnp.float32),
            pltpu.VMEM((OUTROWS * 16,), jnp.float32),
            pltpu.VMEM_SHARED((2 * NSUB * ROW,), jnp.float32),
            pltpu.SemaphoreType.DMA,
        ],
    )
    return f(xs, sc)


def kernel(boxes, scores):
    xs = jnp.zeros((4, P), jnp.float32).at[:, :N].set(boxes.T)
    sc = jnp.full((P,), -1.0, jnp.float32).at[:N].set(scores)
    out = _nms_sc(xs, sc)
    return out.reshape(MAX_DET, 16)[:, :5]
